# Initial kernel scaffold; baseline (speedup 1.0000x reference)
import jax
import jax.numpy as jnp
from jax.experimental import pallas as pl


def kernel(u_feat, g_feat, e_feat, u_idx, g_idx, W1, b1, W2, b2, W3, b3, Wn1, bn1, Wn2, bn2):
    raise NotImplementedError("write your pallas kernel here")


# speedup vs baseline: 9.0262x; 9.0262x over previous
import functools

import jax
import jax.numpy as jnp
from jax import lax
from jax.experimental import pallas as pl
from jax.experimental.pallas import tpu as pltpu
from jax.experimental.pallas import tpu_sc as plsc

NU = 10000
NG = 10000
NE = 320000
NC, NS = 2, 16
NW = NC * NS
EPW = NE // NW
CHUNK = 2000
NCHUNK = EPW // CHUNK
ROWS, COLS = 125, 80
BE = 2560
NBLK = NE // BE

_f32 = jnp.float32
_i32 = jnp.int32

_sc_mesh = plsc.VectorSubcoreMesh(
    core_axis_name="c", subcore_axis_name="s", num_cores=NC, num_subcores=NS)


def _gather_body(uflat_hbm, gflat_hbm, uidx_hbm, gidx_hbm, ef_hbm, x_hbm,
                 utab, gtab, uidx_v, gidx_v, ef_v, x_v):
    wid = lax.axis_index("s") * NC + lax.axis_index("c")
    base = wid * EPW
    pltpu.sync_copy(uflat_hbm, utab)
    pltpu.sync_copy(gflat_hbm, gtab)
    lane = lax.iota(_i32, 16)

    def chunk_body(ci, carry):
        e0 = base + ci * CHUNK
        pltpu.sync_copy(uidx_hbm.at[pl.ds(e0, CHUNK)], uidx_v)
        pltpu.sync_copy(gidx_hbm.at[pl.ds(e0, CHUNK)], gidx_v)
        pltpu.sync_copy(ef_hbm.at[pl.ds(e0 * 2, CHUNK * 2)], ef_v)

        def grp(k, c2):
            iu = uidx_v[pl.ds(k * 16, 16)]
            ig = gidx_v[pl.ds(k * 16, 16)]
            xbase = k * 128 + lane * 8
            for col in range(3):
                vu = plsc.load_gather(utab, [iu * 3 + col])
                plsc.store_scatter(x_v, [xbase + col], vu)
                vg = plsc.load_gather(gtab, [ig * 3 + col])
                plsc.store_scatter(x_v, [xbase + 3 + col], vg)
            for h in range(2):
                ev = ef_v[pl.ds(k * 32 + h * 16, 16)]
                eidx = (k * 16 + h * 8 + (lane >> 1)) * 8 + 6 + (lane & 1)
                plsc.store_scatter(x_v, [eidx], ev)
            return c2

        lax.fori_loop(0, CHUNK // 16, grp, 0)
        pltpu.sync_copy(x_v, x_hbm.at[pl.ds(e0 * 8, CHUNK * 8)])
        return carry

    lax.fori_loop(0, NCHUNK, chunk_body, 0)


_gather_call = functools.partial(
    pl.kernel,
    _gather_body,
    out_type=jax.ShapeDtypeStruct((NE * 8,), _f32),
    mesh=_sc_mesh,
    compiler_params=pltpu.CompilerParams(needs_layout_passes=False),
    scratch_types=[
        pltpu.VMEM((NU * 3,), _f32),
        pltpu.VMEM((NG * 3,), _f32),
        pltpu.VMEM((CHUNK,), _i32),
        pltpu.VMEM((CHUNK,), _i32),
        pltpu.VMEM((CHUNK * 2,), _f32),
        pltpu.VMEM((CHUNK * 8,), _f32),
    ],
)


def _edge_mlp_body(x_ref, w1_ref, b1_ref, w2_ref, b2_ref, w3_ref, b3_ref,
                   out_ref, mx_ref):
    i = pl.program_id(0)
    h = jnp.dot(x_ref[...], w1_ref[...], preferred_element_type=_f32) + b1_ref[...]
    h = jnp.maximum(h, 0.0)
    h = jnp.dot(h, w2_ref[...], preferred_element_type=_f32) + b2_ref[...]
    h = jnp.maximum(h, 0.0)
    logits = jnp.sum(h * w3_ref[...], axis=1, keepdims=True) + b3_ref[...]
    out_ref[...] = logits
    cur = jnp.max(logits, keepdims=True)
    prev = jnp.where(i == 0, jnp.full((1, 1), -jnp.inf, _f32), mx_ref[...])
    mx_ref[...] = jnp.maximum(prev, cur)


def _edge_mlp(x8, w1, b1r, w2, b2r, w3r, b3r):
    return pl.pallas_call(
        _edge_mlp_body,
        grid=(NBLK,),
        in_specs=[
            pl.BlockSpec((BE, 8), lambda i: (i, 0)),
            pl.BlockSpec((8, 256), lambda i: (0, 0)),
            pl.BlockSpec((1, 256), lambda i: (0, 0)),
            pl.BlockSpec((256, 256), lambda i: (0, 0)),
            pl.BlockSpec((1, 256), lambda i: (0, 0)),
            pl.BlockSpec((1, 256), lambda i: (0, 0)),
            pl.BlockSpec((1, 1), lambda i: (0, 0)),
        ],
        out_specs=[
            pl.BlockSpec((BE, 1), lambda i: (i, 0)),
            pl.BlockSpec((1, 1), lambda i: (0, 0)),
        ],
        out_shape=[
            jax.ShapeDtypeStruct((NE, 1), _f32),
            jax.ShapeDtypeStruct((1, 1), _f32),
        ],
    )(x8, w1, b1r, w2, b2r, w3r, b3r)


def _node_mlp_body(u4_ref, wn1_ref, bn1_ref, wn2_ref, bn2_ref, out_ref, mx_ref):
    h = jnp.dot(u4_ref[...], wn1_ref[...], preferred_element_type=_f32) + bn1_ref[...]
    h = jnp.maximum(h, 0.0)
    ln = jnp.sum(h * wn2_ref[...], axis=1, keepdims=True) + bn2_ref[...]
    out_ref[...] = ln
    mx_ref[...] = jnp.max(ln, keepdims=True)


def _node_mlp(u4, wn1, bn1r, wn2r, bn2r):
    return pl.pallas_call(
        _node_mlp_body,
        out_shape=[
            jax.ShapeDtypeStruct((NU, 1), _f32),
            jax.ShapeDtypeStruct((1, 1), _f32),
        ],
    )(u4, wn1, bn1r, wn2r, bn2r)


def _denom_body(el2_hbm, seg3_hbm, m16_hbm, part_hbm, lv, segv, exv, accv, mv,
                shared):
    c = lax.axis_index("c")
    s = lax.axis_index("s")
    wid = s * NC + c
    pltpu.sync_copy(el2_hbm.at[wid], lv)
    pltpu.sync_copy(seg3_hbm.at[wid], segv)
    pltpu.sync_copy(m16_hbm, mv)
    m = mv[...]

    def zero(i, carry):
        accv[pl.ds(i * 16, 16)] = jnp.zeros((16,), _f32)
        return carry

    lax.fori_loop(0, NU // 16, zero, 0)

    @pl.when(s == 0)
    def _():
        pltpu.sync_copy(accv, shared)

    plsc.subcore_barrier()

    def expo(i, carry):
        sl = pl.ds(i * 16, 16)
        exv[sl] = jnp.exp(lv[sl] - m)
        return carry

    lax.fori_loop(0, EPW // 16, expo, 0)

    def srow(i, carry):
        pltpu.sync_copy(exv.at[pl.ds(i * COLS, COLS)], shared.at[segv.at[i]],
                        add=True)
        return carry

    lax.fori_loop(0, ROWS, srow, 0)
    plsc.subcore_barrier()

    @pl.when(s == 0)
    def _():
        pltpu.sync_copy(shared, accv)
        pltpu.sync_copy(accv, part_hbm.at[c])


_denom_call = functools.partial(
    pl.kernel,
    _denom_body,
    out_type=jax.ShapeDtypeStruct((NC, NU), _f32),
    mesh=_sc_mesh,
    compiler_params=pltpu.CompilerParams(needs_layout_passes=False),
    scratch_types=[
        pltpu.VMEM((EPW,), _f32),
        pltpu.VMEM((ROWS, COLS), _i32),
        pltpu.VMEM((EPW,), _f32),
        pltpu.VMEM((NU,), _f32),
        pltpu.VMEM((16,), _f32),
        pltpu.VMEM_SHARED((NU,), _f32),
    ],
)


def _probs_body(el2_hbm, seg2_hbm, ln_hbm, part_hbm, m16_hbm, pe_hbm, pn_hbm,
                lv, segv, denv, tmpv, lnv, pnv, pev, mv):
    c = lax.axis_index("c")
    s = lax.axis_index("s")
    wid = s * NC + c
    pltpu.sync_copy(part_hbm.at[0], denv)
    pltpu.sync_copy(part_hbm.at[1], tmpv)
    pltpu.sync_copy(ln_hbm, lnv)
    pltpu.sync_copy(m16_hbm, mv)
    m = mv[...]

    def dloop(i, carry):
        sl = pl.ds(i * 16, 16)
        en = jnp.exp(lnv[sl] - m)
        d = denv[sl] + tmpv[sl] + en
        denv[sl] = d
        pnv[sl] = en / d
        return carry

    lax.fori_loop(0, NU // 16, dloop, 0)

    @pl.when(wid == 0)
    def _():
        pltpu.sync_copy(pnv, pn_hbm)

    pltpu.sync_copy(el2_hbm.at[wid], lv)
    pltpu.sync_copy(seg2_hbm.at[wid], segv)

    def ploop(i, carry):
        sl = pl.ds(i * 16, 16)
        l = lv[sl]
        sg = segv[sl]
        d = plsc.load_gather(denv, [sg])
        pev[sl] = jnp.exp(l - m) / d
        return carry

    lax.fori_loop(0, EPW // 16, ploop, 0)
    pltpu.sync_copy(pev, pe_hbm.at[wid])


_probs_call = functools.partial(
    pl.kernel,
    _probs_body,
    out_type=[
        jax.ShapeDtypeStruct((NW, EPW), _f32),
        jax.ShapeDtypeStruct((NU,), _f32),
    ],
    mesh=_sc_mesh,
    compiler_params=pltpu.CompilerParams(needs_layout_passes=False),
    scratch_types=[
        pltpu.VMEM((EPW,), _f32),
        pltpu.VMEM((EPW,), _i32),
        pltpu.VMEM((NU,), _f32),
        pltpu.VMEM((NU,), _f32),
        pltpu.VMEM((NU,), _f32),
        pltpu.VMEM((NU,), _f32),
        pltpu.VMEM((EPW,), _f32),
        pltpu.VMEM((16,), _f32),
    ],
)


def kernel(u_feat, g_feat, e_feat, u_idx, g_idx, W1, b1, W2, b2, W3, b3,
           Wn1, bn1, Wn2, bn2):
    uflat = u_feat.reshape(NU * 3)
    gflat = g_feat.reshape(NG * 3)
    eflat = e_feat.reshape(NE * 2)

    xflat = _gather_call()(uflat, gflat, u_idx, g_idx, eflat)
    x8 = xflat.reshape(NE, 8)

    b1r = b1.reshape(1, 256)
    b2r = b2.reshape(1, 256)
    w3r = W3.reshape(1, 256)
    b3r = b3.reshape(1, 1)
    el, me = _edge_mlp(x8, W1, b1r, W2, b2r, w3r, b3r)

    u4 = jnp.pad(u_feat, ((0, 0), (0, 1)))
    wn1 = jnp.pad(Wn1, ((0, 1), (0, 0)))
    bn1r = bn1.reshape(1, 128)
    wn2r = Wn2.reshape(1, 128)
    bn2r = bn2.reshape(1, 1)
    ln, mn = _node_mlp(u4, wn1, bn1r, wn2r, bn2r)

    m16 = jnp.full((16,), jnp.maximum(me[0, 0], mn[0, 0]), _f32)
    el2 = el.reshape(NW, EPW)
    seg3 = u_idx.reshape(NW, ROWS, COLS)
    seg2 = u_idx.reshape(NW, EPW)
    lnf = ln.reshape(NU)

    partials = _denom_call()(el2, seg3, m16)
    pe, pn = _probs_call()(el2, seg2, lnf, partials, m16)

    return jnp.concatenate([pe.reshape(NE), pn])
